# Initial kernel scaffold; baseline (speedup 1.0000x reference)
import jax
import jax.numpy as jnp
from jax.experimental import pallas as pl


def kernel(all_scores, gt_heatmap, keypoints_list):
    raise NotImplementedError("write your pallas kernel here")


# speedup vs baseline: 1.1898x; 1.1898x over previous
import jax
import jax.numpy as jnp
from jax.experimental import pallas as pl
from jax.experimental.pallas import tpu as pltpu


def _loss_body(scores_ref, hm_ref, kp_ref, out_ref):
    B, K, N = scores_ref.shape
    HW = hm_ref.shape[2]
    W = 64
    hw_idx = jax.lax.broadcasted_iota(jnp.int32, (1, HW), 1)
    yrow = (hw_idx // W).astype(jnp.float32)
    xrow = (hw_idx % W).astype(jnp.float32)

    pos_loss = jnp.float32(0.0)
    neg_loss = jnp.float32(0.0)
    neg_count = jnp.float32(0.0)
    for b in range(B):
        ky = kp_ref[b, :, 0:1]
        kx = kp_ref[b, :, 1:2]
        dy = ky - yrow
        dx = kx - xrow
        dist2 = dy * dy + dx * dx
        for n in range(N):
            m = (hm_ref[b, n:n + 1, :] != 0.0)
            masked = jnp.where(m, dist2, jnp.inf)
            d2min = jnp.min(masked, axis=1, keepdims=True)
            d = jnp.sqrt(d2min)
            s = scores_ref[b, :, n:n + 1]
            pos = d < 1.0
            safe_d = jnp.where(pos, d, 0.0)
            safe_s = jnp.where(pos, s, 1.0)
            pos_loss += jnp.sum(
                jnp.where(pos, 10000.0 / (1.0 + jnp.exp(safe_d)) * jnp.log(safe_s), 0.0))
            safe_ns = jnp.where(pos, 0.5, 1.0 - s)
            neg_loss += jnp.sum(jnp.where(pos, 0.0, jnp.log(safe_ns)))
            neg_count += jnp.sum(jnp.logical_not(pos).astype(jnp.float32))
    loss = -pos_loss
    loss = jnp.where(neg_count > 0, loss - 10000.0 / neg_count * neg_loss, loss)
    out_ref[0, 0] = loss


def kernel(all_scores, gt_heatmap, keypoints_list):
    B, K, N = all_scores.shape
    H, W = gt_heatmap.shape[2], gt_heatmap.shape[3]
    hm = gt_heatmap.reshape(B, N, H * W)
    out = pl.pallas_call(
        _loss_body,
        out_shape=jax.ShapeDtypeStruct((1, 1), jnp.float32),
        in_specs=[
            pl.BlockSpec(memory_space=pltpu.VMEM),
            pl.BlockSpec(memory_space=pltpu.VMEM),
            pl.BlockSpec(memory_space=pltpu.VMEM),
        ],
        out_specs=pl.BlockSpec(memory_space=pltpu.SMEM),
    )(all_scores, hm, keypoints_list)
    return out[0, 0]
